# Initial kernel scaffold; baseline (speedup 1.0000x reference)
import jax
import jax.numpy as jnp
from jax.experimental import pallas as pl


def kernel(hidden_states, fast_hcm_state, slow_hcm_state, W_item, b_item, W_query, b_query, W_r1, b_r1, W_r2, b_r2, W_mq, b_mq, W_g, b_g, W_o, b_o, ln_g, ln_b):
    raise NotImplementedError("write your pallas kernel here")


# speedup vs baseline: 18.5392x; 18.5392x over previous
import functools

import jax
import jax.numpy as jnp
from jax.experimental import pallas as pl

B, S, H = 8, 512, 1024
HCM = 512
SLOTS = 64
ALPHA = 0.1
GAMMA = 0.99
ETA = 0.05


def _fused_kernel(hs_ref, fast_ref, slow_ref,
                  w_item_ref, b_item_ref, w_query_ref, b_query_ref,
                  w_r1_ref, b_r1_ref, w_r2_ref, b_r2_ref,
                  w_mq_ref, b_mq_ref,
                  wg_h_ref, wg_r_ref, bg_ref,
                  wo_q_ref, wo_r_ref, bo_ref,
                  ln_g_ref, ln_b_ref,
                  out_ref, newfast_ref, newslow_ref):
    x = hs_ref[0]
    fast0 = fast_ref[0]
    slow0 = slow_ref[0]
    mem = jnp.concatenate([fast0, slow0], axis=0)

    f32 = jnp.float32
    items = jnp.dot(x, w_item_ref[...], preferred_element_type=f32) + b_item_ref[...]
    h1 = jax.nn.gelu(jnp.dot(items, w_r1_ref[...], preferred_element_type=f32) + b_r1_ref[...])
    ub = jnp.dot(h1, w_r2_ref[...], preferred_element_type=f32) + b_r2_ref[...]
    query = jnp.dot(x, w_query_ref[...], preferred_element_type=f32) + b_query_ref[...]
    q_mem = jnp.dot(query, w_mq_ref[...], preferred_element_type=f32) + b_mq_ref[...]
    qk = ub * q_mem

    scale = 1.0 / jnp.sqrt(jnp.float32(HCM))
    scores = jax.lax.dot_general(qk, mem, (((1,), (1,)), ((), ())),
                                 preferred_element_type=f32) * scale
    m = jnp.max(scores, axis=-1, keepdims=True)
    p = jnp.exp(scores - m)
    w = p / jnp.sum(p, axis=-1, keepdims=True)
    retrieved = jnp.dot(w, mem, preferred_element_type=f32) * ub

    fg = jax.nn.sigmoid(jnp.dot(x, wg_h_ref[...], preferred_element_type=f32)
                        + jnp.dot(retrieved, wg_r_ref[...], preferred_element_type=f32)
                        + bg_ref[...])

    t = jax.lax.broadcasted_iota(jnp.int32, (S, 1), 0)
    cons = (t % 10) == 0
    e = jnp.where(cons, 1.0 - ETA, 1.0)
    u = ALPHA * fg
    logA = jnp.log((1.0 - u) * e)
    row = jax.lax.broadcasted_iota(jnp.int32, (S, S), 0)
    col = jax.lax.broadcasted_iota(jnp.int32, (S, S), 1)
    lower = (col <= row).astype(f32)
    L = jnp.dot(lower, logA, preferred_element_type=f32)
    Llast = L[S - 1:S, :]
    ue = u * e
    Cf = ue * jnp.exp(Llast - L)
    nafter = (S - 1) // 10 - t // 10
    wv = jnp.where(cons, (ETA / (1.0 - ETA)) * jnp.exp(nafter.astype(f32) * jnp.log(f32(GAMMA))), 0.0)
    qv = wv * jnp.exp(L)
    wsum = jax.lax.dot_general(lower, qv, (((0,), (0,)), ((), ())),
                               preferred_element_type=f32)
    Cs = ue * wsum * jnp.exp(-L)

    plast_col = jnp.transpose(jnp.exp(Llast))
    w0_col = jnp.transpose(wsum[0:1, :])
    ncons = (S + 9) // 10
    newfast_ref[0] = plast_col * fast0 + jax.lax.dot_general(
        Cf, items, (((0,), (0,)), ((), ())), preferred_element_type=f32)
    newslow_ref[0] = (GAMMA ** ncons) * slow0 + w0_col * fast0 + jax.lax.dot_general(
        Cs, items, (((0,), (0,)), ((), ())), preferred_element_type=f32)

    out = (jnp.dot(query, wo_q_ref[...], preferred_element_type=f32)
           + jnp.dot(retrieved, wo_r_ref[...], preferred_element_type=f32)
           + bo_ref[...])
    y = x + out
    mu = jnp.mean(y, axis=-1, keepdims=True)
    var = jnp.mean((y - mu) ** 2, axis=-1, keepdims=True)
    out_ref[0] = (y - mu) / jnp.sqrt(var + 1e-5) * ln_g_ref[...] + ln_b_ref[...]


@functools.partial(jax.jit, static_argnames=())
def kernel(hidden_states, fast_hcm_state, slow_hcm_state, W_item, b_item,
           W_query, b_query, W_r1, b_r1, W_r2, b_r2, W_mq, b_mq,
           W_g, b_g, W_o, b_o, ln_g, ln_b):
    wg_h = W_g[:H, :SLOTS]
    wg_r = W_g[H:, :SLOTS]
    bg = b_g[:SLOTS].reshape(1, SLOTS)
    wo_q = W_o[:H, :]
    wo_r = W_o[H:, :]

    row2 = lambda v: v.reshape(1, -1)

    full = lambda shp: pl.BlockSpec(shp, lambda b: (0,) * len(shp))
    per_b3 = lambda d0, d1: pl.BlockSpec((1, d0, d1), lambda b: (b, 0, 0))

    out_shapes = (
        jax.ShapeDtypeStruct((B, S, H), jnp.float32),
        jax.ShapeDtypeStruct((B, SLOTS, HCM), jnp.float32),
        jax.ShapeDtypeStruct((B, SLOTS, HCM), jnp.float32),
    )
    return pl.pallas_call(
        _fused_kernel,
        grid=(B,),
        in_specs=[
            per_b3(S, H), per_b3(SLOTS, HCM), per_b3(SLOTS, HCM),
            full((H, HCM)), full((1, HCM)),
            full((H, H)), full((1, H)),
            full((HCM, 2 * HCM)), full((1, 2 * HCM)),
            full((2 * HCM, HCM)), full((1, HCM)),
            full((H, HCM)), full((1, HCM)),
            full((H, SLOTS)), full((HCM, SLOTS)), full((1, SLOTS)),
            full((H, H)), full((HCM, H)), full((1, H)),
            full((1, H)), full((1, H)),
        ],
        out_specs=(per_b3(S, H), per_b3(SLOTS, HCM), per_b3(SLOTS, HCM)),
        out_shape=out_shapes,
    )(hidden_states, fast_hcm_state, slow_hcm_state,
      W_item, row2(b_item), W_query, row2(b_query),
      W_r1, row2(b_r1), W_r2, row2(b_r2),
      W_mq, row2(b_mq),
      wg_h, wg_r, bg,
      wo_q, wo_r, row2(b_o),
      row2(ln_g), row2(ln_b))


# speedup vs baseline: 21.3481x; 1.1515x over previous
import functools

import jax
import jax.numpy as jnp
from jax.experimental import pallas as pl
from jax.experimental.pallas import tpu as pltpu

B, S, H = 8, 512, 1024
HCM = 512
SLOTS = 64
ALPHA = 0.1
GAMMA = 0.99
ETA = 0.05

_BF = jnp.bfloat16
_F32 = jnp.float32


def _fused_kernel(hs_ref, fast_ref, slow_ref,
                  w_item_ref, b_item_ref, w_query_ref, b_query_ref,
                  w_r1_ref, b_r1_ref, w_r2_ref, b_r2_ref,
                  w_mq_ref, b_mq_ref,
                  w_g_ref, b_g_ref,
                  w_o_ref, b_o_ref,
                  ln_g_ref, ln_b_ref,
                  out_ref, newfast_ref, newslow_ref,
                  w_item_b, w_query_b, w_r1_b, w_r2_b, w_mq_b,
                  wg_h_b, wg_r_b, wo_q_b, wo_r_b):
    b = pl.program_id(0)

    @pl.when(b == 0)
    def _cast_weights():
        w_item_b[...] = w_item_ref[...].astype(_BF)
        w_query_b[...] = w_query_ref[...].astype(_BF)
        w_r1_b[...] = w_r1_ref[...].astype(_BF)
        w_r2_b[...] = w_r2_ref[...].astype(_BF)
        w_mq_b[...] = w_mq_ref[...].astype(_BF)
        wg_h_b[...] = w_g_ref[0:H, 0:SLOTS].astype(_BF)
        wg_r_b[...] = w_g_ref[H:H + HCM, 0:SLOTS].astype(_BF)
        wo_q_b[...] = w_o_ref[0:H, :].astype(_BF)
        wo_r_b[...] = w_o_ref[H:H + HCM, :].astype(_BF)

    x = hs_ref[0]
    xb = x.astype(_BF)
    fast0 = fast_ref[0]
    slow0 = slow_ref[0]
    mem_b = jnp.concatenate([fast0, slow0], axis=0).astype(_BF)

    items = jnp.dot(xb, w_item_b[...], preferred_element_type=_F32) + b_item_ref[...]
    h1 = jax.nn.gelu(jnp.dot(items.astype(_BF), w_r1_b[...], preferred_element_type=_F32) + b_r1_ref[...])
    ub = jnp.dot(h1.astype(_BF), w_r2_b[...], preferred_element_type=_F32) + b_r2_ref[...]
    query = jnp.dot(xb, w_query_b[...], preferred_element_type=_F32) + b_query_ref[...]
    q_mem = jnp.dot(query.astype(_BF), w_mq_b[...], preferred_element_type=_F32) + b_mq_ref[...]
    qk = ub * q_mem

    scale = 1.0 / jnp.sqrt(jnp.float32(HCM))
    scores = jax.lax.dot_general(qk.astype(_BF), mem_b, (((1,), (1,)), ((), ())),
                                 preferred_element_type=_F32) * scale
    m = jnp.max(scores, axis=-1, keepdims=True)
    p = jnp.exp(scores - m)
    w = p / jnp.sum(p, axis=-1, keepdims=True)
    retrieved = jnp.dot(w.astype(_BF), mem_b, preferred_element_type=_F32) * ub
    retr_b = retrieved.astype(_BF)

    fg = jax.nn.sigmoid(jnp.dot(xb, wg_h_b[...], preferred_element_type=_F32)
                        + jnp.dot(retr_b, wg_r_b[...], preferred_element_type=_F32)
                        + b_g_ref[0:1, 0:SLOTS])

    t = jax.lax.broadcasted_iota(jnp.int32, (S, 1), 0)
    cons = (t % 10) == 0
    e = jnp.where(cons, 1.0 - ETA, 1.0)
    u = ALPHA * fg
    logA = jnp.log((1.0 - u) * e)
    row = jax.lax.broadcasted_iota(jnp.int32, (S, S), 0)
    col = jax.lax.broadcasted_iota(jnp.int32, (S, S), 1)
    lower = (col <= row).astype(_F32)
    L = jnp.dot(lower, logA, preferred_element_type=_F32)
    Llast = L[S - 1:S, :]
    ue = u * e
    Cf = ue * jnp.exp(Llast - L)
    nafter = (S - 1) // 10 - t // 10
    wv = jnp.where(cons, (ETA / (1.0 - ETA)) * jnp.exp(nafter.astype(_F32) * jnp.log(_F32(GAMMA))), 0.0)
    qv = wv * jnp.exp(L)
    wsum = jax.lax.dot_general(lower, qv, (((0,), (0,)), ((), ())),
                               preferred_element_type=_F32)
    Cs = ue * wsum * jnp.exp(-L)

    plast_col = jnp.transpose(jnp.exp(Llast))
    w0_col = jnp.transpose(wsum[0:1, :])
    ncons = (S + 9) // 10
    newfast_ref[0] = plast_col * fast0 + jax.lax.dot_general(
        Cf, items, (((0,), (0,)), ((), ())), preferred_element_type=_F32)
    newslow_ref[0] = (GAMMA ** ncons) * slow0 + w0_col * fast0 + jax.lax.dot_general(
        Cs, items, (((0,), (0,)), ((), ())), preferred_element_type=_F32)

    out = (jnp.dot(query.astype(_BF), wo_q_b[...], preferred_element_type=_F32)
           + jnp.dot(retr_b, wo_r_b[...], preferred_element_type=_F32)
           + b_o_ref[...])
    y = x + out
    mu = jnp.mean(y, axis=-1, keepdims=True)
    var = jnp.mean((y - mu) ** 2, axis=-1, keepdims=True)
    out_ref[0] = (y - mu) / jnp.sqrt(var + 1e-5) * ln_g_ref[...] + ln_b_ref[...]


@functools.partial(jax.jit, static_argnames=())
def kernel(hidden_states, fast_hcm_state, slow_hcm_state, W_item, b_item,
           W_query, b_query, W_r1, b_r1, W_r2, b_r2, W_mq, b_mq,
           W_g, b_g, W_o, b_o, ln_g, ln_b):
    row2 = lambda v: v.reshape(1, -1)

    full = lambda shp: pl.BlockSpec(shp, lambda b: (0,) * len(shp))
    per_b3 = lambda d0, d1: pl.BlockSpec((1, d0, d1), lambda b: (b, 0, 0))

    out_shapes = (
        jax.ShapeDtypeStruct((B, S, H), jnp.float32),
        jax.ShapeDtypeStruct((B, SLOTS, HCM), jnp.float32),
        jax.ShapeDtypeStruct((B, SLOTS, HCM), jnp.float32),
    )
    return pl.pallas_call(
        _fused_kernel,
        grid=(B,),
        in_specs=[
            per_b3(S, H), per_b3(SLOTS, HCM), per_b3(SLOTS, HCM),
            full((H, HCM)), full((1, HCM)),
            full((H, H)), full((1, H)),
            full((HCM, 2 * HCM)), full((1, 2 * HCM)),
            full((2 * HCM, HCM)), full((1, HCM)),
            full((H, HCM)), full((1, HCM)),
            full((H + HCM, 2 * SLOTS)), full((1, 2 * SLOTS)),
            full((H + HCM, H)), full((1, H)),
            full((1, H)), full((1, H)),
        ],
        out_specs=(per_b3(S, H), per_b3(SLOTS, HCM), per_b3(SLOTS, HCM)),
        out_shape=out_shapes,
        scratch_shapes=[
            pltpu.VMEM((H, HCM), _BF), pltpu.VMEM((H, H), _BF),
            pltpu.VMEM((HCM, 2 * HCM), _BF), pltpu.VMEM((2 * HCM, HCM), _BF),
            pltpu.VMEM((H, HCM), _BF),
            pltpu.VMEM((H, SLOTS), _BF), pltpu.VMEM((HCM, SLOTS), _BF),
            pltpu.VMEM((H, H), _BF), pltpu.VMEM((HCM, H), _BF),
        ],
    )(hidden_states, fast_hcm_state, slow_hcm_state,
      W_item, row2(b_item), W_query, row2(b_query),
      W_r1, row2(b_r1), W_r2, row2(b_r2),
      W_mq, row2(b_mq),
      W_g, row2(b_g),
      W_o, row2(b_o),
      row2(ln_g), row2(ln_b))


# speedup vs baseline: 21.6709x; 1.0151x over previous
import functools

import jax
import jax.numpy as jnp
from jax.experimental import pallas as pl
from jax.experimental.pallas import tpu as pltpu

B, S, H = 8, 512, 1024
HCM = 512
SLOTS = 64
ALPHA = 0.1
GAMMA = 0.99
ETA = 0.05
BB = 2

_BF = jnp.bfloat16
_F32 = jnp.float32


def _fused_kernel(hs_ref, fast_ref, slow_ref,
                  w_item_ref, b_item_ref, w_query_ref, b_query_ref,
                  w_r1_ref, b_r1_ref, w_r2_ref, b_r2_ref,
                  w_mq_ref, b_mq_ref,
                  w_g_ref, b_g_ref,
                  w_o_ref, b_o_ref,
                  ln_g_ref, ln_b_ref,
                  out_ref, newfast_ref, newslow_ref,
                  w_item_b, w_query_b, w_r1_b, w_r2_b, w_mq_b,
                  wg_h_b, wg_r_b, wo_q_b, wo_r_b):
    step = pl.program_id(0)

    @pl.when(step == 0)
    def _cast_weights():
        w_item_b[...] = w_item_ref[...].astype(_BF)
        w_query_b[...] = w_query_ref[...].astype(_BF)
        w_r1_b[...] = w_r1_ref[...].astype(_BF)
        w_r2_b[...] = w_r2_ref[...].astype(_BF)
        w_mq_b[...] = w_mq_ref[...].astype(_BF)
        wg_h_b[...] = w_g_ref[0:H, 0:SLOTS].astype(_BF)
        wg_r_b[...] = w_g_ref[H:H + HCM, 0:SLOTS].astype(_BF)
        wo_q_b[...] = w_o_ref[0:H, :].astype(_BF)
        wo_r_b[...] = w_o_ref[H:H + HCM, :].astype(_BF)

    x2 = hs_ref[...].reshape(BB * S, H)
    xb = x2.astype(_BF)

    items2 = jnp.dot(xb, w_item_b[...], preferred_element_type=_F32) + b_item_ref[...]
    h12 = jax.nn.gelu(jnp.dot(items2.astype(_BF), w_r1_b[...], preferred_element_type=_F32) + b_r1_ref[...])
    ub2 = jnp.dot(h12.astype(_BF), w_r2_b[...], preferred_element_type=_F32) + b_r2_ref[...]
    query2 = jnp.dot(xb, w_query_b[...], preferred_element_type=_F32) + b_query_ref[...]
    q_mem2 = jnp.dot(query2.astype(_BF), w_mq_b[...], preferred_element_type=_F32) + b_mq_ref[...]
    qk2 = ub2 * q_mem2

    scale = 1.0 / jnp.sqrt(jnp.float32(HCM))
    retr = []
    mems = []
    for i in range(BB):
        mem_b = jnp.concatenate([fast_ref[i], slow_ref[i]], axis=0).astype(_BF)
        mems.append(mem_b)
        qk = qk2[i * S:(i + 1) * S]
        scores = jax.lax.dot_general(qk.astype(_BF), mem_b, (((1,), (1,)), ((), ())),
                                     preferred_element_type=_F32) * scale
        m = jnp.max(scores, axis=-1, keepdims=True)
        p = jnp.exp(scores - m)
        w = p / jnp.sum(p, axis=-1, keepdims=True)
        retr.append(jnp.dot(w.astype(_BF), mem_b, preferred_element_type=_F32))
    retrieved2 = jnp.concatenate(retr, axis=0) * ub2
    retr_b = retrieved2.astype(_BF)

    fg2 = jax.nn.sigmoid(jnp.dot(xb, wg_h_b[...], preferred_element_type=_F32)
                         + jnp.dot(retr_b, wg_r_b[...], preferred_element_type=_F32)
                         + b_g_ref[0:1, 0:SLOTS])

    t = jax.lax.broadcasted_iota(jnp.int32, (S, 1), 0)
    cons = (t % 10) == 0
    e = jnp.where(cons, 1.0 - ETA, 1.0)
    row = jax.lax.broadcasted_iota(jnp.int32, (S, S), 0)
    col = jax.lax.broadcasted_iota(jnp.int32, (S, S), 1)
    lower = (col <= row).astype(_F32)
    nafter = (S - 1) // 10 - t // 10
    wv = jnp.where(cons, (ETA / (1.0 - ETA)) * jnp.exp(nafter.astype(_F32) * jnp.log(_F32(GAMMA))), 0.0)
    ncons = (S + 9) // 10

    for i in range(BB):
        fast0 = fast_ref[i]
        slow0 = slow_ref[i]
        items = items2[i * S:(i + 1) * S]
        u = ALPHA * fg2[i * S:(i + 1) * S]
        logA = jnp.log((1.0 - u) * e)
        L = jnp.dot(lower, logA, preferred_element_type=_F32)
        Llast = L[S - 1:S, :]
        ue = u * e
        Cf = ue * jnp.exp(Llast - L)
        qv = wv * jnp.exp(L)
        wsum = jax.lax.dot_general(lower, qv, (((0,), (0,)), ((), ())),
                                   preferred_element_type=_F32)
        Cs = ue * wsum * jnp.exp(-L)
        plast_col = jnp.transpose(jnp.exp(Llast))
        w0_col = jnp.transpose(wsum[0:1, :])
        newfast_ref[i] = plast_col * fast0 + jax.lax.dot_general(
            Cf, items, (((0,), (0,)), ((), ())), preferred_element_type=_F32)
        newslow_ref[i] = (GAMMA ** ncons) * slow0 + w0_col * fast0 + jax.lax.dot_general(
            Cs, items, (((0,), (0,)), ((), ())), preferred_element_type=_F32)

    out2 = (jnp.dot(query2.astype(_BF), wo_q_b[...], preferred_element_type=_F32)
            + jnp.dot(retr_b, wo_r_b[...], preferred_element_type=_F32)
            + b_o_ref[...])
    y = x2 + out2
    mu = jnp.mean(y, axis=-1, keepdims=True)
    var = jnp.mean((y - mu) ** 2, axis=-1, keepdims=True)
    out_ref[...] = ((y - mu) / jnp.sqrt(var + 1e-5) * ln_g_ref[...] + ln_b_ref[...]).reshape(BB, S, H)


@functools.partial(jax.jit, static_argnames=())
def kernel(hidden_states, fast_hcm_state, slow_hcm_state, W_item, b_item,
           W_query, b_query, W_r1, b_r1, W_r2, b_r2, W_mq, b_mq,
           W_g, b_g, W_o, b_o, ln_g, ln_b):
    row2 = lambda v: v.reshape(1, -1)

    full = lambda shp: pl.BlockSpec(shp, lambda b: (0,) * len(shp))
    per_b3 = lambda d0, d1: pl.BlockSpec((BB, d0, d1), lambda b: (b, 0, 0))

    out_shapes = (
        jax.ShapeDtypeStruct((B, S, H), jnp.float32),
        jax.ShapeDtypeStruct((B, SLOTS, HCM), jnp.float32),
        jax.ShapeDtypeStruct((B, SLOTS, HCM), jnp.float32),
    )
    return pl.pallas_call(
        _fused_kernel,
        grid=(B // BB,),
        in_specs=[
            per_b3(S, H), per_b3(SLOTS, HCM), per_b3(SLOTS, HCM),
            full((H, HCM)), full((1, HCM)),
            full((H, H)), full((1, H)),
            full((HCM, 2 * HCM)), full((1, 2 * HCM)),
            full((2 * HCM, HCM)), full((1, HCM)),
            full((H, HCM)), full((1, HCM)),
            full((H + HCM, 2 * SLOTS)), full((1, 2 * SLOTS)),
            full((H + HCM, H)), full((1, H)),
            full((1, H)), full((1, H)),
        ],
        out_specs=(per_b3(S, H), per_b3(SLOTS, HCM), per_b3(SLOTS, HCM)),
        out_shape=out_shapes,
        scratch_shapes=[
            pltpu.VMEM((H, HCM), _BF), pltpu.VMEM((H, H), _BF),
            pltpu.VMEM((HCM, 2 * HCM), _BF), pltpu.VMEM((2 * HCM, HCM), _BF),
            pltpu.VMEM((H, HCM), _BF),
            pltpu.VMEM((H, SLOTS), _BF), pltpu.VMEM((HCM, SLOTS), _BF),
            pltpu.VMEM((H, H), _BF), pltpu.VMEM((HCM, H), _BF),
        ],
    )(hidden_states, fast_hcm_state, slow_hcm_state,
      W_item, row2(b_item), W_query, row2(b_query),
      W_r1, row2(b_r1), W_r2, row2(b_r2),
      W_mq, row2(b_mq),
      W_g, row2(b_g),
      W_o, row2(b_o),
      row2(ln_g), row2(ln_b))
